# baseline (speedup 1.0000x reference)
import functools

import jax
import jax.numpy as jnp
from jax import lax
from jax.experimental import pallas as pl
from jax.experimental.pallas import tpu as pltpu
from jax.experimental.pallas import tpu_sc as plsc

VOCAB = 32000
HIDDEN = 768
BATCH = 4096
SEQ = 200

NC = 2
NS = 16
NW = NC * NS
B_PER_W = BATCH // NW

NCHUNK = 5
CHUNK = SEQ // NCHUNK
HGRP = HIDDEN // 16
SCALE = 1.0 / SEQ

_mesh = plsc.VectorSubcoreMesh(core_axis_name="c", subcore_axis_name="s")


@functools.partial(
    pl.kernel,
    out_type=jax.ShapeDtypeStruct((BATCH, HIDDEN), jnp.float32),
    mesh=_mesh,
    scratch_types=[
        pltpu.VMEM((B_PER_W * SEQ,), jnp.int32),
        pltpu.VMEM((2, CHUNK, HIDDEN), jnp.float32),
        pltpu.VMEM((HIDDEN,), jnp.float32),
        pltpu.SemaphoreType.DMA,
        pltpu.SemaphoreType.DMA,
    ],
)
def _pool_kernel(ids_hbm, table_hbm, out_hbm, idx_v, rows_v, acc_v, sem0, sem1):
    wid = lax.axis_index("s") * NC + lax.axis_index("c")
    base = wid * B_PER_W
    sems = (sem0, sem1)

    pltpu.sync_copy(ids_hbm.at[pl.ds(base * SEQ, B_PER_W * SEQ)], idx_v)

    def accum(buf, c):
        def h_body(h, _):
            hh = h * 16

            @plsc.parallel_loop(0, CHUNK, carry=jnp.zeros((16,), jnp.float32),
                                unroll=4)
            def chunk_sum(k, acc):
                return acc + rows_v[buf, k, pl.ds(hh, 16)]

            if c == 0:
                acc_v[pl.ds(hh, 16)] = chunk_sum
            elif c < NCHUNK - 1:
                acc_v[pl.ds(hh, 16)] = acc_v[pl.ds(hh, 16)] + chunk_sum
            else:
                acc_v[pl.ds(hh, 16)] = (acc_v[pl.ds(hh, 16)] + chunk_sum) * SCALE
            return 0

        lax.fori_loop(0, HGRP, h_body, 0)

    def idx_slice(r, c):
        off = pl.multiple_of(r * SEQ + c * CHUNK, CHUNK)
        return idx_v.at[pl.ds(off, CHUNK)]

    def row_body(r, _):
        b = base + r
        copies = [None, None]
        copies[0] = pltpu.async_copy(
            table_hbm.at[idx_slice(r, 0)], rows_v.at[0], sems[0])
        for c in range(NCHUNK):
            nxt = (c + 1) % 2
            if c + 1 < NCHUNK:
                copies[nxt] = pltpu.async_copy(
                    table_hbm.at[idx_slice(r, c + 1)], rows_v.at[nxt], sems[nxt])
            copies[c % 2].wait()
            accum(c % 2, c)
        pltpu.sync_copy(acc_v, out_hbm.at[b])
        return 0

    lax.fori_loop(0, B_PER_W, row_body, 0)


def _mm_body(x_ref, w_ref, b_ref, o_ref):
    o_ref[...] = (
        jnp.dot(x_ref[...], w_ref[...], preferred_element_type=jnp.float32)
        + b_ref[...]
    )


def _matmul(pooled, W, b):
    M, K = pooled.shape
    N = W.shape[1]
    BM = 2048
    BN = 1280
    return pl.pallas_call(
        _mm_body,
        grid=(M // BM, N // BN),
        in_specs=[
            pl.BlockSpec((BM, K), lambda i, j: (i, 0)),
            pl.BlockSpec((K, BN), lambda i, j: (0, j)),
            pl.BlockSpec((1, BN), lambda i, j: (0, j)),
        ],
        out_specs=pl.BlockSpec((BM, BN), lambda i, j: (i, j)),
        out_shape=jax.ShapeDtypeStruct((M, N), jnp.float32),
    )(pooled, W, b.reshape(1, N))


@jax.jit
def kernel(input_ids, embedding, W, b):
    ids = input_ids.astype(jnp.int32).reshape(BATCH * SEQ)
    pooled = _pool_kernel(ids, embedding)
    return _matmul(pooled, W, b)


# speedup vs baseline: 3.2886x; 3.2886x over previous
import functools

import jax
import jax.numpy as jnp
from jax import lax
from jax.experimental import pallas as pl
from jax.experimental.pallas import tpu as pltpu
from jax.experimental.pallas import tpu_sc as plsc

VOCAB = 32000
HIDDEN = 768
BATCH = 4096
SEQ = 200

NC = 2
NS = 16
NW = NC * NS
B_PER_W = BATCH // NW

NCHUNK = 5
CHUNK = SEQ // NCHUNK
HGRP = HIDDEN // 16
SCALE = 1.0 / SEQ

_mesh = plsc.VectorSubcoreMesh(core_axis_name="c", subcore_axis_name="s")


@functools.partial(
    pl.kernel,
    out_type=jax.ShapeDtypeStruct((BATCH, HIDDEN), jnp.float32),
    mesh=_mesh,
    scratch_types=[
        pltpu.VMEM((B_PER_W * SEQ,), jnp.int32),
        pltpu.VMEM((2, CHUNK, HIDDEN), jnp.float32),
        pltpu.VMEM((HIDDEN,), jnp.float32),
        pltpu.SemaphoreType.DMA,
        pltpu.SemaphoreType.DMA,
    ],
)
def _pool_kernel(ids_hbm, table_hbm, out_hbm, idx_v, rows_v, acc_v, sem0, sem1):
    wid = lax.axis_index("s") * NC + lax.axis_index("c")
    base = wid * B_PER_W
    sems = (sem0, sem1)

    pltpu.sync_copy(ids_hbm.at[pl.ds(base * SEQ, B_PER_W * SEQ)], idx_v)

    def accum(buf, c):
        def h_body(h, _):
            hh = pl.multiple_of(h * 16, 16)
            sl = pl.ds(hh, 16)
            a = [rows_v[buf, j, sl] for j in range(4)]
            for k in range(4, CHUNK, 4):
                for j in range(4):
                    a[j] = a[j] + rows_v[buf, k + j, sl]
            chunk_sum = (a[0] + a[1]) + (a[2] + a[3])
            if c == 0:
                acc_v[sl] = chunk_sum
            elif c < NCHUNK - 1:
                acc_v[sl] = acc_v[sl] + chunk_sum
            else:
                acc_v[sl] = (acc_v[sl] + chunk_sum) * SCALE
            return 0

        lax.fori_loop(0, HGRP, h_body, 0)

    def idx_slice(r, c):
        off = pl.multiple_of(r * SEQ + c * CHUNK, CHUNK)
        return idx_v.at[pl.ds(off, CHUNK)]

    def row_body(r, _):
        b = base + r
        copies = [None, None]
        copies[0] = pltpu.async_copy(
            table_hbm.at[idx_slice(r, 0)], rows_v.at[0], sems[0])
        for c in range(NCHUNK):
            nxt = (c + 1) % 2
            if c + 1 < NCHUNK:
                copies[nxt] = pltpu.async_copy(
                    table_hbm.at[idx_slice(r, c + 1)], rows_v.at[nxt], sems[nxt])
            copies[c % 2].wait()
            accum(c % 2, c)
        pltpu.sync_copy(acc_v, out_hbm.at[b])
        return 0

    lax.fori_loop(0, B_PER_W, row_body, 0)


def _mm_body(x_ref, w_ref, b_ref, o_ref):
    o_ref[...] = (
        jnp.dot(x_ref[...], w_ref[...], preferred_element_type=jnp.float32)
        + b_ref[...]
    )


def _matmul(pooled, W, b):
    M, K = pooled.shape
    N = W.shape[1]
    BM = 2048
    BN = 1280
    return pl.pallas_call(
        _mm_body,
        grid=(M // BM, N // BN),
        in_specs=[
            pl.BlockSpec((BM, K), lambda i, j: (i, 0)),
            pl.BlockSpec((K, BN), lambda i, j: (0, j)),
            pl.BlockSpec((1, BN), lambda i, j: (0, j)),
        ],
        out_specs=pl.BlockSpec((BM, BN), lambda i, j: (i, j)),
        out_shape=jax.ShapeDtypeStruct((M, N), jnp.float32),
    )(pooled, W, b.reshape(1, N))


@jax.jit
def kernel(input_ids, embedding, W, b):
    ids = input_ids.astype(jnp.int32).reshape(BATCH * SEQ)
    pooled = _pool_kernel(ids, embedding)
    return _matmul(pooled, W, b)
